# baseline (speedup 1.0000x reference)
import functools

import numpy as np
import jax
import jax.numpy as jnp
from jax import lax
from jax.experimental import pallas as pl
from jax.experimental.pallas import tpu as pltpu
from jax.experimental.pallas import tpu_sc as plsc

NC = 2
NS = 16
L = 16
NW = NC * NS
BL = 128

NUM_MODS = 6
NUM_PREDS = 30


def _splat_i(x):
    return jnp.full((L,), x, dtype=jnp.int32)


def _sc_body(num_rows, reg_hbm, gt_hbm, has_hbm, loss_out, cnt_out,
             reg_v0, gt_v0, has_v0, reg_v1, gt_v1, has_v1,
             stage_f, stage_i, sem0, sem1):
    nblk = num_rows // BL
    blk_per_w = nblk // NW
    wid = lax.axis_index("s") * NC + lax.axis_index("c")
    lanes = lax.iota(jnp.int32, L)
    c_np = (np.float32(0.1) * np.arange(NUM_PREDS, dtype=np.float32)
            / np.float32(NUM_PREDS))
    zero_i = _splat_i(0)
    one_i = _splat_i(1)
    zero = jnp.zeros((L,), jnp.float32)
    bufs = ((reg_v0, gt_v0, has_v0, sem0), (reg_v1, gt_v1, has_v1, sem1))

    def start_block(q, buf):
        reg_v, gt_v, has_v, sem = buf
        pltpu.async_copy(reg_hbm.at[:, :, pl.ds(2 * q, 2), :], reg_v, sem)
        pltpu.async_copy(gt_hbm.at[:, pl.ds(2 * q, 2), :], gt_v, sem)
        pltpu.async_copy(has_hbm.at[:, pl.ds(q, 1), :], has_v, sem)

    def wait_block(buf):
        reg_v, gt_v, has_v, sem = buf
        pltpu.make_async_copy(reg_hbm.at[:, :, pl.ds(0, 2), :], reg_v, sem).wait()
        pltpu.make_async_copy(gt_hbm.at[:, pl.ds(0, 2), :], gt_v, sem).wait()
        pltpu.make_async_copy(has_hbm.at[:, pl.ds(0, 1), :], has_v, sem).wait()

    def make_block_compute(buf):
        reg_v4, gt_v3, has_v, _ = buf

        def gather_flat(ref, flat_idx):
            zeros = [zero_i] * (len(ref.shape) - 1)
            return plsc.load_gather(ref, zeros + [flat_idx])

        def subgroup(s, carry2):
            acc_loss, acc_cnt = carry2
            off = s * L
            vlanes = lanes + off

            h = has_v[0, 0, pl.ds(off, L)]
            best = h + c_np[0]
            hsum = h
            for j in range(1, NUM_PREDS):
                h = has_v[j, 0, pl.ds(off, L)]
                best = jnp.maximum(best, h + c_np[j])
                hsum = hsum + h
            maskf = jnp.where(best > 1.0, jnp.float32(1.0), jnp.float32(0.0))
            hasany = jnp.where(best >= 1.0, jnp.float32(1.0), jnp.float32(0.0))
            bidx = ((best - hasany) * jnp.float32(10.0 * NUM_PREDS)
                    + jnp.float32(0.5)).astype(jnp.int32)

            base_e = bidx * (2 * BL) + vlanes
            gx = gather_flat(gt_v3, base_e)
            gy = gather_flat(gt_v3, base_e + BL)
            dbest = None
            midx = zero_i
            mstride = NUM_PREDS * 2 * BL
            for m in range(NUM_MODS):
                ex = gather_flat(reg_v4, base_e + (m * mstride))
                ey = gather_flat(reg_v4, base_e + (m * mstride + BL))
                dx = ex - gx
                dy = ey - gy
                d = dx * dx + dy * dy
                if m == 0:
                    dbest = d
                else:
                    p = d < dbest
                    dbest = jnp.where(p, d, dbest)
                    midx = jnp.where(p, _splat_i(m), midx)

            base_l = midx * mstride + vlanes
            for j in range(NUM_PREDS):
                sm = has_v[j, 0, pl.ds(off, L)] * maskf
                for c in range(2):
                    r = gather_flat(reg_v4, base_l + (j * 2 * BL + c * BL))
                    t = gt_v3[j, c, pl.ds(off, L)]
                    d = (r - t) * sm
                    ad = jnp.abs(d)
                    w = jnp.where(ad < 1.0, jnp.float32(0.5) * d * d,
                                  ad - jnp.float32(0.5))
                    acc_loss = acc_loss + w
            acc_cnt = acc_cnt + hsum * maskf
            return acc_loss, acc_cnt

        return subgroup

    assert blk_per_w % 2 == 0
    q0 = wid * blk_per_w
    qend = q0 + blk_per_w
    start_block(q0, bufs[0])
    start_block(q0 + 1, bufs[1])

    def pair(t, carry):
        acc = carry
        q = q0 + 2 * t
        for k in range(2):
            wait_block(bufs[k])
            acc = plsc.parallel_loop(
                0, BL // L, 1, unroll=2, carry=acc)(make_block_compute(bufs[k]))

            @pl.when(q + 2 + k < qend)
            def _():
                start_block(q + 2 + k, bufs[k])
        return acc

    acc_loss, acc_cnt = lax.fori_loop(0, blk_per_w // 2, pair, (zero, zero))
    stage_f[...] = acc_loss
    pltpu.sync_copy(stage_f, loss_out.at[pl.ds(wid * L, L)])
    stage_i[...] = acc_cnt.astype(jnp.int32)
    pltpu.sync_copy(stage_i, cnt_out.at[pl.ds(wid * L, L)])


def kernel(reg, gt_preds, has_preds):
    n = reg.shape[0]
    assert n % (BL * NW) == 0
    nblk = n // BL
    reg_y = (reg.transpose(1, 2, 0, 3)
             .reshape(NUM_MODS, NUM_PREDS, nblk, BL, 2)
             .transpose(0, 1, 2, 4, 3)
             .reshape(NUM_MODS, NUM_PREDS, 2 * nblk, BL))
    gt_y = (gt_preds.transpose(1, 2, 0)
            .reshape(NUM_PREDS, 2, nblk, BL)
            .transpose(0, 2, 1, 3)
            .reshape(NUM_PREDS, 2 * nblk, BL))
    has_y = has_preds.T.astype(jnp.float32).reshape(NUM_PREDS, nblk, BL)
    mesh = plsc.VectorSubcoreMesh(core_axis_name="c", subcore_axis_name="s")
    run = pl.kernel(
        functools.partial(_sc_body, n),
        out_type=(
            jax.ShapeDtypeStruct((NW * L,), jnp.float32),
            jax.ShapeDtypeStruct((NW * L,), jnp.int32),
        ),
        mesh=mesh,
        compiler_params=pltpu.CompilerParams(needs_layout_passes=False),
        scratch_types=[
            pltpu.VMEM((NUM_MODS, NUM_PREDS, 2, BL), jnp.float32),
            pltpu.VMEM((NUM_PREDS, 2, BL), jnp.float32),
            pltpu.VMEM((NUM_PREDS, 1, BL), jnp.float32),
            pltpu.VMEM((NUM_MODS, NUM_PREDS, 2, BL), jnp.float32),
            pltpu.VMEM((NUM_PREDS, 2, BL), jnp.float32),
            pltpu.VMEM((NUM_PREDS, 1, BL), jnp.float32),
            pltpu.VMEM((L,), jnp.float32),
            pltpu.VMEM((L,), jnp.int32),
            pltpu.SemaphoreType.DMA,
            pltpu.SemaphoreType.DMA,
        ],
    )
    loss_p, cnt_p = run(reg_y, gt_y, has_y)
    reg_loss = loss_p.sum()
    num_reg = cnt_p.sum()
    return (reg_loss, num_reg)


# speedup vs baseline: 1.0040x; 1.0040x over previous
import functools

import numpy as np
import jax
import jax.numpy as jnp
from jax import lax
from jax.experimental import pallas as pl
from jax.experimental.pallas import tpu as pltpu
from jax.experimental.pallas import tpu_sc as plsc

NC = 2
NS = 16
L = 16
NW = NC * NS
BL = 128

NUM_MODS = 6
NUM_PREDS = 30


def _splat_i(x):
    return jnp.full((L,), x, dtype=jnp.int32)


def _sc_body(num_rows, reg_hbm, gt_hbm, has_hbm, loss_out, cnt_out,
             reg_v0, gt_v0, has_v0, reg_v1, gt_v1, has_v1,
             stage_f, stage_i, sem0, sem1):
    nblk = num_rows // BL
    blk_per_w = nblk // NW
    wid = lax.axis_index("s") * NC + lax.axis_index("c")
    lanes = lax.iota(jnp.int32, L)
    c_np = (np.float32(0.1) * np.arange(NUM_PREDS, dtype=np.float32)
            / np.float32(NUM_PREDS))
    zero_i = _splat_i(0)
    zero = jnp.zeros((L,), jnp.float32)
    bufs = ((reg_v0, gt_v0, has_v0, sem0), (reg_v1, gt_v1, has_v1, sem1))

    def start_block(q, buf):
        reg_v, gt_v, has_v, sem = buf
        pltpu.async_copy(reg_hbm.at[:, :, pl.ds(2 * q, 2), :], reg_v, sem)
        pltpu.async_copy(gt_hbm.at[:, pl.ds(2 * q, 2), :], gt_v, sem)
        pltpu.async_copy(has_hbm.at[:, pl.ds(q, 1), :], has_v, sem)

    def wait_block(buf):
        reg_v, gt_v, has_v, sem = buf
        pltpu.make_async_copy(reg_hbm.at[:, :, pl.ds(0, 2), :], reg_v, sem).wait()
        pltpu.make_async_copy(gt_hbm.at[:, pl.ds(0, 2), :], gt_v, sem).wait()
        pltpu.make_async_copy(has_hbm.at[:, pl.ds(0, 1), :], has_v, sem).wait()

    def make_block_compute(buf):
        reg_v4, gt_v3, has_v, _ = buf

        def gather_flat(ref, flat_idx):
            zeros = [zero_i] * (len(ref.shape) - 1)
            return plsc.load_gather(ref, zeros + [flat_idx])

        def subgroup(s, carry2):
            acc_loss, acc_cnt = carry2
            off = s * L
            vlanes = lanes + off

            h = has_v[0, 0, pl.ds(off, L)]
            best = h + c_np[0]
            hsum = h
            for j in range(1, NUM_PREDS):
                h = has_v[j, 0, pl.ds(off, L)]
                best = jnp.maximum(best, h + c_np[j])
                hsum = hsum + h
            maskf = jnp.where(best > 1.0, jnp.float32(1.0), jnp.float32(0.0))
            hasany = jnp.where(best >= 1.0, jnp.float32(1.0), jnp.float32(0.0))
            bidx = ((best - hasany) * jnp.float32(10.0 * NUM_PREDS)
                    + jnp.float32(0.5)).astype(jnp.int32)

            base_e = bidx * (2 * BL) + vlanes
            gx = gather_flat(gt_v3, base_e)
            gy = gather_flat(gt_v3, base_e + BL)
            dbest = None
            midx = zero_i
            mstride = NUM_PREDS * 2 * BL
            for m in range(NUM_MODS):
                ex = gather_flat(reg_v4, base_e + (m * mstride))
                ey = gather_flat(reg_v4, base_e + (m * mstride + BL))
                dx = ex - gx
                dy = ey - gy
                d = dx * dx + dy * dy
                if m == 0:
                    dbest = d
                else:
                    p = d < dbest
                    dbest = jnp.where(p, d, dbest)
                    midx = jnp.where(p, _splat_i(m), midx)

            base_l = midx * mstride + vlanes
            for j in range(NUM_PREDS):
                sm = has_v[j, 0, pl.ds(off, L)] * maskf
                for c in range(2):
                    r = gather_flat(reg_v4, base_l + (j * 2 * BL + c * BL))
                    t = gt_v3[j, c, pl.ds(off, L)]
                    d = (r - t) * sm
                    ad = jnp.abs(d)
                    w = jnp.where(ad < 1.0, jnp.float32(0.5) * d * d,
                                  ad - jnp.float32(0.5))
                    acc_loss = acc_loss + w
            acc_cnt = acc_cnt + hsum * maskf
            return acc_loss, acc_cnt

        return subgroup

    assert blk_per_w % 2 == 0
    q0 = wid * blk_per_w
    qend = q0 + blk_per_w
    start_block(q0, bufs[0])
    start_block(q0 + 1, bufs[1])

    def pair(t, carry):
        acc = carry
        q = q0 + 2 * t
        for k in range(2):
            wait_block(bufs[k])
            acc = plsc.parallel_loop(
                0, BL // L, 1, unroll=2, carry=acc)(make_block_compute(bufs[k]))

            @pl.when(q + 2 + k < qend)
            def _():
                start_block(q + 2 + k, bufs[k])
        return acc

    acc_loss, acc_cnt = lax.fori_loop(0, blk_per_w // 2, pair, (zero, zero))
    stage_f[...] = acc_loss
    pltpu.sync_copy(stage_f, loss_out.at[pl.ds(wid * L, L)])
    stage_i[...] = acc_cnt.astype(jnp.int32)
    pltpu.sync_copy(stage_i, cnt_out.at[pl.ds(wid * L, L)])


def kernel(reg, gt_preds, has_preds):
    n = reg.shape[0]
    assert n % (BL * NW) == 0
    nblk = n // BL
    reg_y = (reg.transpose(1, 2, 0, 3)
             .reshape(NUM_MODS, NUM_PREDS, nblk, BL, 2)
             .transpose(0, 1, 2, 4, 3)
             .reshape(NUM_MODS, NUM_PREDS, 2 * nblk, BL))
    gt_y = (gt_preds.transpose(1, 2, 0)
            .reshape(NUM_PREDS, 2, nblk, BL)
            .transpose(0, 2, 1, 3)
            .reshape(NUM_PREDS, 2 * nblk, BL))
    has_y = has_preds.T.astype(jnp.float32).reshape(NUM_PREDS, nblk, BL)
    mesh = plsc.VectorSubcoreMesh(core_axis_name="c", subcore_axis_name="s")
    run = pl.kernel(
        functools.partial(_sc_body, n),
        out_type=(
            jax.ShapeDtypeStruct((NW * L,), jnp.float32),
            jax.ShapeDtypeStruct((NW * L,), jnp.int32),
        ),
        mesh=mesh,
        compiler_params=pltpu.CompilerParams(needs_layout_passes=False),
        scratch_types=[
            pltpu.VMEM((NUM_MODS, NUM_PREDS, 2, BL), jnp.float32),
            pltpu.VMEM((NUM_PREDS, 2, BL), jnp.float32),
            pltpu.VMEM((NUM_PREDS, 1, BL), jnp.float32),
            pltpu.VMEM((NUM_MODS, NUM_PREDS, 2, BL), jnp.float32),
            pltpu.VMEM((NUM_PREDS, 2, BL), jnp.float32),
            pltpu.VMEM((NUM_PREDS, 1, BL), jnp.float32),
            pltpu.VMEM((L,), jnp.float32),
            pltpu.VMEM((L,), jnp.int32),
            pltpu.SemaphoreType.DMA,
            pltpu.SemaphoreType.DMA,
        ],
    )
    loss_p, cnt_p = run(reg_y, gt_y, has_y)
    reg_loss = loss_p.sum()
    num_reg = cnt_p.sum()
    return (reg_loss, num_reg)
